# baseline (speedup 1.0000x reference)
import functools

import jax
import jax.numpy as jnp
from jax import lax
from jax.experimental import pallas as pl
from jax.experimental.pallas import tpu as pltpu
from jax.experimental.pallas import tpu_sc as plsc

N = 10000
NP = 10240
E = 160000
EP = 163840
EPR = EP // 128
D_IN = 256
D_HID = 256
D_OUT = 128

_MESH = plsc.VectorSubcoreMesh(
    core_axis_name="c", subcore_axis_name="s", num_cores=2, num_subcores=16)


def _mm1_body(x_ref, w_ref, o_ref):
    o_ref[...] = lax.dot_general(
        x_ref[...], w_ref[...], (((1,), (0,)), ((), ())),
        precision=lax.Precision.DEFAULT, preferred_element_type=jnp.float32)


def _matmul1(x_p, W1):
    BN = 2560
    nb = NP // BN
    return pl.pallas_call(
        _mm1_body,
        grid=(nb, 2),
        in_specs=[
            pl.BlockSpec((BN, D_IN), lambda i, c: (i, 0)),
            pl.BlockSpec((D_IN, 128), lambda i, c: (0, c)),
        ],
        out_specs=pl.BlockSpec((BN, 128), lambda i, c: (c * nb + i, 0)),
        out_shape=jax.ShapeDtypeStruct((2 * NP, 128), jnp.float32),
    )(x_p, W1)


def _mm2_body(a_ref, b_ref, b1a_ref, b1b_ref, w2a_ref, w2b_ref, o_ref):
    ga = jnp.maximum(a_ref[...] + b1a_ref[0, 0], 0.0)
    gb = jnp.maximum(b_ref[...] + b1b_ref[0, 0], 0.0)
    oa = lax.dot_general(ga, w2a_ref[0], (((1,), (0,)), ((), ())),
                         precision=lax.Precision.DEFAULT,
                         preferred_element_type=jnp.float32)
    ob = lax.dot_general(gb, w2b_ref[0], (((1,), (0,)), ((), ())),
                         precision=lax.Precision.DEFAULT,
                         preferred_element_type=jnp.float32)
    o_ref[...] = oa + ob


def _matmul2(a1cat, b1r, W2r):
    BN = 2560
    nb = NP // BN
    return pl.pallas_call(
        _mm2_body,
        grid=(nb,),
        in_specs=[
            pl.BlockSpec((BN, 128), lambda i: (i, 0)),
            pl.BlockSpec((BN, 128), lambda i: (nb + i, 0)),
            pl.BlockSpec((1, 1, 128), lambda i: (0, 0, 0)),
            pl.BlockSpec((1, 1, 128), lambda i: (1, 0, 0)),
            pl.BlockSpec((1, 128, 128), lambda i: (0, 0, 0)),
            pl.BlockSpec((1, 128, 128), lambda i: (1, 0, 0)),
        ],
        out_specs=pl.BlockSpec((BN, 128), lambda i: (i, 0)),
        out_shape=jax.ShapeDtypeStruct((NP, 128), jnp.float32),
    )(a1cat, a1cat, b1r, b1r, W2r, W2r)


def _sum_body(p0_ref, p1_ref, b2_ref, o_ref):
    o_ref[...] = p0_ref[...] + p1_ref[...] + b2_ref[0, 0]


def _sum_tc(parts, b2r):
    BN = 2560
    nb = NP // BN
    return pl.pallas_call(
        _sum_body,
        grid=(nb,),
        in_specs=[
            pl.BlockSpec((BN, 128), lambda i: (i, 0)),
            pl.BlockSpec((BN, 128), lambda i: (nb + i, 0)),
            pl.BlockSpec((1, 1, 128), lambda i: (0, 0, 0)),
        ],
        out_specs=pl.BlockSpec((BN, 128), lambda i: (i, 0)),
        out_shape=jax.ShapeDtypeStruct((NP, 128), jnp.float32),
    )(parts, parts, b2r)


@functools.partial(
    pl.kernel,
    out_type=jax.ShapeDtypeStruct((EP,), jnp.float32),
    mesh=_MESH,
    compiler_params=pltpu.CompilerParams(needs_layout_passes=False),
    scratch_types=[
        pltpu.VMEM_SHARED((NP,), jnp.float32),
        pltpu.VMEM((80, 128), jnp.int32),
        pltpu.VMEM((80, 128), jnp.int32),
        pltpu.VMEM((EP // 16,), jnp.float32),
        pltpu.VMEM((640,), jnp.float32),
        pltpu.VMEM((NP,), jnp.float32),
        pltpu.SemaphoreType.DMA,
        pltpu.SemaphoreType.DMA,
    ],
)
def _norm_kernel(dstm_h, src1_h, w1_h, norm_h, deg_s, dstv, srcv, wv,
                 degv, disv, dsem, psem):
    c = lax.axis_index("c")
    s = lax.axis_index("s")
    ept = EP // 16

    z16 = jnp.zeros((16,), jnp.float32)

    def _zero(i, carry):
        degv[pl.ds(i * 16, 16)] = z16
        return carry
    lax.fori_loop(0, 40, _zero, 0)
    pltpu.sync_copy(degv, deg_s.at[pl.ds(s * 640, 640)])
    plsc.subcore_barrier()

    pltpu.sync_copy(
        dstm_h.at[pl.ds(pl.multiple_of(s * 80, 8), 80)], dstv)
    pltpu.sync_copy(w1_h.at[pl.ds(s * ept, ept)], wv)
    pltpu.make_async_copy(
        src1_h.at[pl.ds(pl.multiple_of(s * 80, 8), 80)], srcv, psem).start()

    def _acc(j, carry):
        pltpu.async_copy(wv.at[pl.ds(j * 128, 128)],
                         deg_s.at[dstv.at[j]], dsem, add=True)
        return carry
    lax.fori_loop(0, 80, _acc, 0)

    def _drain(j, carry):
        pltpu.make_async_copy(wv.at[pl.ds(j * 128, 128)],
                              deg_s.at[dstv.at[j]], dsem).wait()
        return carry
    lax.fori_loop(0, 80, _drain, 0)
    plsc.subcore_barrier()

    pltpu.sync_copy(deg_s.at[pl.ds(s * 640, 640)], degv)

    def _rsqrt(i, carry):
        d = degv[pl.ds(i * 16, 16)]
        dsafe = jnp.maximum(d, 1e-30)
        y = 0.25 * dsafe + 1.0
        for _ in range(12):
            y = 0.5 * (y + dsafe / y)
        r = 1.0 / y
        degv[pl.ds(i * 16, 16)] = jnp.where(d > 0.0, r, 0.0)
        return carry
    lax.fori_loop(0, 40, _rsqrt, 0)
    plsc.subcore_barrier()
    pltpu.sync_copy(degv, deg_s.at[pl.ds(s * 640, 640)])
    plsc.subcore_barrier()

    pltpu.sync_copy(deg_s, disv)
    pltpu.make_async_copy(
        src1_h.at[pl.ds(pl.multiple_of(s * 80, 8), 80)], srcv, psem).wait()

    def _nrm(r, carry):
        for g in range(8):
            off = r * 128 + g * 16
            s16 = srcv[r, pl.ds(g * 16, 16)]
            d16 = dstv[r, pl.ds(g * 16, 16)]
            gs = plsc.load_gather(disv, [s16])
            gd = plsc.load_gather(disv, [d16])
            wv[pl.ds(off, 16)] = gs * wv[pl.ds(off, 16)] * gd
        return carry
    lax.fori_loop(0, 80, _nrm, 0)

    @pl.when(c == 0)
    def _():
        pltpu.sync_copy(wv, norm_h.at[pl.ds(s * ept, ept)])


def _make_agg(col_split):
    eh = EP // 32
    NCH = eh // 40

    scratch = [
        pltpu.VMEM_SHARED((NP, 128), jnp.float32),
        pltpu.VMEM((eh,), jnp.int32),
        pltpu.VMEM((eh,), jnp.float32),
        pltpu.VMEM((NCH, 40), jnp.int32),
        pltpu.VMEM((40, 128), jnp.float32),
        pltpu.VMEM((40, 128), jnp.float32),
        pltpu.VMEM((40, 128), jnp.float32),
        pltpu.VMEM((40, 128), jnp.float32),
        pltpu.SemaphoreType.DMA,
        pltpu.SemaphoreType.DMA,
        pltpu.SemaphoreType.DMA,
        pltpu.SemaphoreType.DMA,
        pltpu.SemaphoreType.DMA,
        pltpu.SemaphoreType.DMA,
        pltpu.SemaphoreType.DMA,
        pltpu.SemaphoreType.DMA,
    ]

    def body(h_h, src1_h, dstm_h, norm1_h, out_h,
             acc, srcv, normv, dstv, b0, b1, b2, b3,
             gs0, gs1, gs2, gs3, ss0, ss1, ss2, ss3):
        c = lax.axis_index("c")
        s = lax.axis_index("s")
        coff = c * NP if col_split else c * 0
        bufs = (b0, b1, b2, b3)
        gsems = (gs0, gs1, gs2, gs3)
        ssems = (ss0, ss1, ss2, ss3)

        z16 = jnp.zeros((16,), jnp.float32)
        for i in range(16):
            for g in range(8):
                b0[i, pl.ds(g * 16, 16)] = z16
        for k in range(40):
            pltpu.sync_copy(b0.at[pl.ds(0, 16)],
                            acc.at[pl.ds(s * 640 + k * 16, 16)])
        plsc.subcore_barrier()

        def _g_start(j, k):
            pltpu.make_async_copy(
                h_h.at[srcv.at[pl.ds(j * 40, 40)]], bufs[k], gsems[k]).start()

        def _g_wait(j, k):
            pltpu.make_async_copy(
                h_h.at[srcv.at[pl.ds(j * 40, 40)]], bufs[k], gsems[k]).wait()

        def _s_start(j, k):
            pltpu.async_copy(bufs[k], acc.at[dstv.at[j]], ssems[k], add=True)

        def _s_wait(j, k):
            pltpu.make_async_copy(
                bufs[k], acc.at[dstv.at[j]], ssems[k]).wait()

        def _scale(j, k):
            rows = bufs[k]

            def _rowpair(r, carry):
                for m in range(2):
                    rr = 2 * r + m
                    nsp = plsc.load_gather(
                        normv, [jnp.full((16,), j * 40 + rr, jnp.int32)])
                    for g in range(8):
                        rows[rr, pl.ds(g * 16, 16)] = (
                            rows[rr, pl.ds(g * 16, 16)] * nsp)
                return carry
            lax.fori_loop(0, 20, _rowpair, 0)

        for p in range(2 if col_split else 1):
            if col_split:
                be = s * (EP // 16) + p * eh
            else:
                be = c * (EP // 2) + s * eh
            bd = pl.multiple_of(be // 40, 8)
            pltpu.sync_copy(src1_h.at[pl.ds(be, eh)], srcv)
            pltpu.sync_copy(norm1_h.at[pl.ds(be, eh)], normv)
            pltpu.sync_copy(dstm_h.at[pl.ds(bd, NCH)], dstv)

            if col_split:
                def _off(r, carry):
                    for g in range(8):
                        o = r * 128 + g * 16
                        srcv[pl.ds(o, 16)] = srcv[pl.ds(o, 16)] + coff
                    return carry
                lax.fori_loop(0, 40, _off, 0)

            _g_start(0, 0)
            _g_start(1, 1)

            def _quad(i, carry):
                for m in range(4):
                    j = 4 * i + m
                    k = m
                    kk = (m + 2) % 4
                    _g_wait(j, k)
                    _scale(j, k)
                    _s_start(j, k)
                    if m < 2:
                        @pl.when(i > 0)
                        def _():
                            _s_wait(j - 2, kk)
                        _g_start(j + 2, kk)
                    else:
                        _s_wait(j - 2, kk)

                        @pl.when(i < NCH // 4 - 1)
                        def _():
                            _g_start(j + 2, kk)
                return carry
            lax.fori_loop(0, NCH // 4, _quad, 0)
            _s_wait(NCH - 2, 2)
            _s_wait(NCH - 1, 3)
        plsc.subcore_barrier()

        for k in range(5):
            r0 = s * 640 + k * 128
            pltpu.sync_copy(acc.at[pl.ds(r0, 128)],
                            out_h.at[pl.ds(c * NP + r0, 128)])

    return pl.kernel(
        body,
        out_type=jax.ShapeDtypeStruct((2 * NP, 128), jnp.float32),
        mesh=_MESH,
        scratch_types=scratch,
        compiler_params=pltpu.CompilerParams(needs_layout_passes=False),
    )


_agg1 = _make_agg(col_split=True)
_agg2 = _make_agg(col_split=False)


def kernel(x, edge_index, edge_weight, W1, b1, W2, b2):
    src = edge_index[0].astype(jnp.int32)
    dst = edge_index[1].astype(jnp.int32)
    npad = EP - E
    pad_idx = (jnp.arange(npad, dtype=jnp.int32) * 37) % N
    src1 = jnp.concatenate([src, pad_idx])
    dst1 = jnp.concatenate([dst, pad_idx])
    w1 = jnp.concatenate([edge_weight, jnp.zeros((npad,), jnp.float32)])
    dstm = dst1.reshape(EP // 40, 40)
    x_p = jnp.pad(x, ((0, NP - N), (0, 0)))
    b1r = b1.reshape(2, 1, 128)
    W2r = W2.reshape(2, 128, 128)
    b2r = b2.reshape(1, 1, 128)

    dstm128 = dst1.reshape(EPR, 128)
    srcm128 = src1.reshape(EPR, 128)
    h1cat = _matmul1(x_p, W1)
    norm1 = _norm_kernel(dstm128, srcm128, w1)
    a1cat = _agg1(h1cat, src1, dstm, norm1)
    h2 = _matmul2(a1cat, b1r, W2r)
    parts2 = _agg2(h2, src1, dstm, norm1)
    z = _sum_tc(parts2, b2r)
    return z[:N]


# speedup vs baseline: 1.0140x; 1.0140x over previous
import functools

import jax
import jax.numpy as jnp
from jax import lax
from jax.experimental import pallas as pl
from jax.experimental.pallas import tpu as pltpu
from jax.experimental.pallas import tpu_sc as plsc

N = 10000
NP = 10240
E = 160000
EP = 163840
EPR = EP // 128
D_IN = 256
D_HID = 256
D_OUT = 128

_MESH = plsc.VectorSubcoreMesh(
    core_axis_name="c", subcore_axis_name="s", num_cores=2, num_subcores=16)


def _mm1_body(x_ref, w_ref, o_ref):
    o_ref[...] = lax.dot_general(
        x_ref[...], w_ref[...], (((1,), (0,)), ((), ())),
        precision=lax.Precision.DEFAULT, preferred_element_type=jnp.float32)


def _matmul1(x, W1):
    BN = 2560
    nb = NP // BN
    return pl.pallas_call(
        _mm1_body,
        grid=(nb, 2),
        in_specs=[
            pl.BlockSpec((BN, D_IN), lambda i, c: (i, 0)),
            pl.BlockSpec((D_IN, 128), lambda i, c: (0, c)),
        ],
        out_specs=pl.BlockSpec((BN, 128), lambda i, c: (c * nb + i, 0)),
        out_shape=jax.ShapeDtypeStruct((2 * NP, 128), jnp.float32),
    )(x, W1)


def _mm2_body(a_ref, b_ref, b1a_ref, b1b_ref, w2a_ref, w2b_ref, o_ref):
    ga = jnp.maximum(a_ref[...] + b1a_ref[0, 0], 0.0)
    gb = jnp.maximum(b_ref[...] + b1b_ref[0, 0], 0.0)
    oa = lax.dot_general(ga, w2a_ref[0], (((1,), (0,)), ((), ())),
                         precision=lax.Precision.DEFAULT,
                         preferred_element_type=jnp.float32)
    ob = lax.dot_general(gb, w2b_ref[0], (((1,), (0,)), ((), ())),
                         precision=lax.Precision.DEFAULT,
                         preferred_element_type=jnp.float32)
    o_ref[...] = oa + ob


def _matmul2(a1cat, b1r, W2r):
    BN = 2560
    nb = NP // BN
    return pl.pallas_call(
        _mm2_body,
        grid=(nb,),
        in_specs=[
            pl.BlockSpec((BN, 128), lambda i: (i, 0)),
            pl.BlockSpec((BN, 128), lambda i: (nb + i, 0)),
            pl.BlockSpec((1, 1, 128), lambda i: (0, 0, 0)),
            pl.BlockSpec((1, 1, 128), lambda i: (1, 0, 0)),
            pl.BlockSpec((1, 128, 128), lambda i: (0, 0, 0)),
            pl.BlockSpec((1, 128, 128), lambda i: (1, 0, 0)),
        ],
        out_specs=pl.BlockSpec((BN, 128), lambda i: (i, 0)),
        out_shape=jax.ShapeDtypeStruct((NP, 128), jnp.float32),
    )(a1cat, a1cat, b1r, b1r, W2r, W2r)


def _sum_body(p0_ref, p1_ref, b2_ref, o_ref):
    o_ref[...] = p0_ref[...] + p1_ref[...] + b2_ref[0, 0]


def _sum_tc(parts, b2r):
    BN = 2560
    nb = NP // BN
    return pl.pallas_call(
        _sum_body,
        grid=(nb,),
        in_specs=[
            pl.BlockSpec((BN, 128), lambda i: (i, 0)),
            pl.BlockSpec((BN, 128), lambda i: (nb + i, 0)),
            pl.BlockSpec((1, 1, 128), lambda i: (0, 0, 0)),
        ],
        out_specs=pl.BlockSpec((BN, 128), lambda i: (i, 0)),
        out_shape=jax.ShapeDtypeStruct((NP, 128), jnp.float32),
    )(parts, parts, b2r)


@functools.partial(
    pl.kernel,
    out_type=jax.ShapeDtypeStruct((EP,), jnp.float32),
    mesh=_MESH,
    compiler_params=pltpu.CompilerParams(needs_layout_passes=False),
    scratch_types=[
        pltpu.VMEM_SHARED((NP,), jnp.float32),
        pltpu.VMEM((80, 128), jnp.int32),
        pltpu.VMEM((80, 128), jnp.int32),
        pltpu.VMEM((EP // 16,), jnp.float32),
        pltpu.VMEM((640,), jnp.float32),
        pltpu.VMEM((NP,), jnp.float32),
        pltpu.SemaphoreType.DMA,
        pltpu.SemaphoreType.DMA,
    ],
)
def _norm_kernel(dstm_h, src1_h, w1_h, norm_h, deg_s, dstv, srcv, wv,
                 degv, disv, dsem, psem):
    c = lax.axis_index("c")
    s = lax.axis_index("s")
    ept = EP // 16

    z16 = jnp.zeros((16,), jnp.float32)

    def _zero(i, carry):
        degv[pl.ds(i * 16, 16)] = z16
        return carry
    lax.fori_loop(0, 40, _zero, 0)
    pltpu.sync_copy(degv, deg_s.at[pl.ds(s * 640, 640)])
    plsc.subcore_barrier()

    pltpu.sync_copy(
        dstm_h.at[pl.ds(pl.multiple_of(s * 80, 8), 80)], dstv)
    pltpu.sync_copy(w1_h.at[pl.ds(s * ept, ept)], wv)
    pltpu.make_async_copy(
        src1_h.at[pl.ds(pl.multiple_of(s * 80, 8), 80)], srcv, psem).start()

    def _acc(j, carry):
        pltpu.async_copy(wv.at[pl.ds(j * 128, 128)],
                         deg_s.at[dstv.at[j]], dsem, add=True)
        return carry
    lax.fori_loop(0, 80, _acc, 0)

    def _drain(j, carry):
        pltpu.make_async_copy(wv.at[pl.ds(j * 128, 128)],
                              deg_s.at[dstv.at[j]], dsem).wait()
        return carry
    lax.fori_loop(0, 80, _drain, 0)
    plsc.subcore_barrier()

    pltpu.sync_copy(deg_s.at[pl.ds(s * 640, 640)], degv)

    def _rsqrt(i, carry):
        d = degv[pl.ds(i * 16, 16)]
        dsafe = jnp.maximum(d, 1e-30)
        y = 0.25 * dsafe + 1.0
        for _ in range(12):
            y = 0.5 * (y + dsafe / y)
        r = 1.0 / y
        degv[pl.ds(i * 16, 16)] = jnp.where(d > 0.0, r, 0.0)
        return carry
    lax.fori_loop(0, 40, _rsqrt, 0)
    plsc.subcore_barrier()
    pltpu.sync_copy(degv, deg_s.at[pl.ds(s * 640, 640)])
    plsc.subcore_barrier()

    pltpu.sync_copy(deg_s, disv)
    pltpu.make_async_copy(
        src1_h.at[pl.ds(pl.multiple_of(s * 80, 8), 80)], srcv, psem).wait()

    def _nrm(r, carry):
        for g in range(8):
            off = r * 128 + g * 16
            s16 = srcv[r, pl.ds(g * 16, 16)]
            d16 = dstv[r, pl.ds(g * 16, 16)]
            gs = plsc.load_gather(disv, [s16])
            gd = plsc.load_gather(disv, [d16])
            wv[pl.ds(off, 16)] = gs * wv[pl.ds(off, 16)] * gd
        return carry
    lax.fori_loop(0, 80, _nrm, 0)

    @pl.when(c == 0)
    def _():
        pltpu.sync_copy(wv, norm_h.at[pl.ds(s * ept, ept)])


def _make_agg(col_split):
    eh = EP // 32
    NCH = eh // 40

    scratch = [
        pltpu.VMEM_SHARED((NP, 128), jnp.float32),
        pltpu.VMEM((eh,), jnp.int32),
        pltpu.VMEM((eh,), jnp.float32),
        pltpu.VMEM((NCH, 40), jnp.int32),
        pltpu.VMEM((40, 128), jnp.float32),
        pltpu.VMEM((40, 128), jnp.float32),
        pltpu.VMEM((40, 128), jnp.float32),
        pltpu.VMEM((40, 128), jnp.float32),
        pltpu.SemaphoreType.DMA,
        pltpu.SemaphoreType.DMA,
        pltpu.SemaphoreType.DMA,
        pltpu.SemaphoreType.DMA,
        pltpu.SemaphoreType.DMA,
        pltpu.SemaphoreType.DMA,
        pltpu.SemaphoreType.DMA,
        pltpu.SemaphoreType.DMA,
    ]

    def body(h_h, src1_h, dstm_h, norm1_h, out_h,
             acc, srcv, normv, dstv, b0, b1, b2, b3,
             gs0, gs1, gs2, gs3, ss0, ss1, ss2, ss3):
        c = lax.axis_index("c")
        s = lax.axis_index("s")
        coff = c * NP if col_split else c * 0
        bufs = (b0, b1, b2, b3)
        gsems = (gs0, gs1, gs2, gs3)
        ssems = (ss0, ss1, ss2, ss3)

        z16 = jnp.zeros((16,), jnp.float32)
        for i in range(16):
            for g in range(8):
                b0[i, pl.ds(g * 16, 16)] = z16
        for k in range(40):
            pltpu.sync_copy(b0.at[pl.ds(0, 16)],
                            acc.at[pl.ds(s * 640 + k * 16, 16)])
        plsc.subcore_barrier()

        def _g_start(j, k):
            pltpu.make_async_copy(
                h_h.at[srcv.at[pl.ds(j * 40, 40)]], bufs[k], gsems[k]).start()

        def _g_wait(j, k):
            pltpu.make_async_copy(
                h_h.at[srcv.at[pl.ds(j * 40, 40)]], bufs[k], gsems[k]).wait()

        def _s_start(j, k):
            pltpu.async_copy(bufs[k], acc.at[dstv.at[j]], ssems[k], add=True)

        def _s_wait(j, k):
            pltpu.make_async_copy(
                bufs[k], acc.at[dstv.at[j]], ssems[k]).wait()

        def _scale(j, k):
            rows = bufs[k]

            def _rowpair(r, carry):
                for m in range(2):
                    rr = 2 * r + m
                    nsp = plsc.load_gather(
                        normv, [jnp.full((16,), j * 40 + rr, jnp.int32)])
                    for g in range(8):
                        rows[rr, pl.ds(g * 16, 16)] = (
                            rows[rr, pl.ds(g * 16, 16)] * nsp)
                return carry
            lax.fori_loop(0, 20, _rowpair, 0)

        for p in range(2 if col_split else 1):
            if col_split:
                be = s * (EP // 16) + p * eh
            else:
                be = c * (EP // 2) + s * eh
            bd = pl.multiple_of(be // 40, 8)
            pltpu.sync_copy(src1_h.at[pl.ds(be, eh)], srcv)
            pltpu.sync_copy(norm1_h.at[pl.ds(be, eh)], normv)
            pltpu.sync_copy(dstm_h.at[pl.ds(bd, NCH)], dstv)

            if col_split:
                def _off(r, carry):
                    for g in range(8):
                        o = r * 128 + g * 16
                        srcv[pl.ds(o, 16)] = srcv[pl.ds(o, 16)] + coff
                    return carry
                lax.fori_loop(0, 40, _off, 0)

            _g_start(0, 0)
            _g_start(1, 1)

            def _quad(i, carry):
                for m in range(4):
                    j = 4 * i + m
                    k = m
                    kk = (m + 2) % 4
                    _g_wait(j, k)
                    _scale(j, k)
                    _s_start(j, k)
                    if m < 2:
                        @pl.when(i > 0)
                        def _():
                            _s_wait(j - 2, kk)
                        _g_start(j + 2, kk)
                    else:
                        _s_wait(j - 2, kk)

                        @pl.when(i < NCH // 4 - 1)
                        def _():
                            _g_start(j + 2, kk)
                return carry
            lax.fori_loop(0, NCH // 4, _quad, 0)
            _s_wait(NCH - 2, 2)
            _s_wait(NCH - 1, 3)
        plsc.subcore_barrier()

        for k in range(5):
            r0 = s * 640 + k * 128
            pltpu.sync_copy(acc.at[pl.ds(r0, 128)],
                            out_h.at[pl.ds(c * NP + r0, 128)])

    return pl.kernel(
        body,
        out_type=jax.ShapeDtypeStruct((2 * NP, 128), jnp.float32),
        mesh=_MESH,
        scratch_types=scratch,
        compiler_params=pltpu.CompilerParams(needs_layout_passes=False),
    )


_agg1 = _make_agg(col_split=True)
_agg2 = _make_agg(col_split=False)


def kernel(x, edge_index, edge_weight, W1, b1, W2, b2):
    src = edge_index[0].astype(jnp.int32)
    dst = edge_index[1].astype(jnp.int32)
    npad = EP - E
    pad_idx = (jnp.arange(npad, dtype=jnp.int32) * 37) % N
    src1 = jnp.concatenate([src, pad_idx])
    dst1 = jnp.concatenate([dst, pad_idx])
    w1 = jnp.concatenate([edge_weight, jnp.zeros((npad,), jnp.float32)])
    dstm = dst1.reshape(EP // 40, 40)
    b1r = b1.reshape(2, 1, 128)
    W2r = W2.reshape(2, 128, 128)
    b2r = b2.reshape(1, 1, 128)

    dstm128 = dst1.reshape(EPR, 128)
    srcm128 = src1.reshape(EPR, 128)
    h1cat = _matmul1(x, W1)
    norm1 = _norm_kernel(dstm128, srcm128, w1)
    a1cat = _agg1(h1cat, src1, dstm, norm1)
    h2 = _matmul2(a1cat, b1r, W2r)
    parts2 = _agg2(h2, src1, dstm, norm1)
    z = _sum_tc(parts2, b2r)
    return z[:N]
